# baseline (speedup 1.0000x reference)
import functools

import jax
import jax.numpy as jnp
from jax import lax
from jax.experimental import pallas as pl
from jax.experimental.pallas import tpu as pltpu
from jax.experimental.pallas import tpu_sc as plsc

_NUM_WORKERS = 32
_CHUNK = 800


def kernel(x, table):
    batch, seq = x.shape
    _, d_model = table.shape
    n = batch * seq
    idx = x.reshape(n).astype(jnp.int32)

    per_worker = n // _NUM_WORKERS
    n_chunks = per_worker // _CHUNK
    assert per_worker * _NUM_WORKERS == n and n_chunks * _CHUNK == per_worker

    mesh = plsc.VectorSubcoreMesh(core_axis_name="c", subcore_axis_name="s")

    @functools.partial(
        pl.kernel,
        out_type=jax.ShapeDtypeStruct((n, d_model), jnp.float32),
        mesh=mesh,
        compiler_params=pltpu.CompilerParams(use_tc_tiling_on_sc=False),
        scratch_types=[
            pltpu.VMEM((_CHUNK,), jnp.int32),
            pltpu.VMEM((_CHUNK, d_model), jnp.float32),
            pltpu.SemaphoreType.DMA,
        ],
    )
    def emb(idx_hbm, table_hbm, out_hbm, idx_v, rows_v, sem):
        wid = lax.axis_index("s") * 2 + lax.axis_index("c")
        base = wid * per_worker

        @pl.loop(0, n_chunks)
        def _(g):
            off = base + g * _CHUNK
            pltpu.sync_copy(idx_hbm.at[pl.ds(off, _CHUNK)], idx_v)
            pltpu.async_copy(table_hbm.at[idx_v], rows_v, sem).wait()
            pltpu.sync_copy(rows_v, out_hbm.at[pl.ds(off, _CHUNK)])

    out = emb(idx, table)
    return out.reshape(batch, seq, d_model)


# speedup vs baseline: 1.0246x; 1.0246x over previous
import functools

import jax
import jax.numpy as jnp
from jax import lax
from jax.experimental import pallas as pl
from jax.experimental.pallas import tpu as pltpu
from jax.experimental.pallas import tpu_sc as plsc

_NUM_WORKERS = 32
_CHUNK = 400
_NBUF = 4
_GLAG = 2


def kernel(x, table):
    batch, seq = x.shape
    _, d_model = table.shape
    n = batch * seq
    idx = x.reshape(n).astype(jnp.int32)

    per_worker = n // _NUM_WORKERS
    n_chunks = per_worker // _CHUNK
    assert per_worker * _NUM_WORKERS == n and n_chunks * _CHUNK == per_worker
    assert n_chunks % _NBUF == 0 and n_chunks > 2 * _NBUF

    mesh = plsc.VectorSubcoreMesh(core_axis_name="c", subcore_axis_name="s")

    @functools.partial(
        pl.kernel,
        out_type=jax.ShapeDtypeStruct((n, d_model), jnp.float32),
        mesh=mesh,
        compiler_params=pltpu.CompilerParams(use_tc_tiling_on_sc=False),
        scratch_types=[
            pltpu.VMEM((per_worker,), jnp.int32),
            pltpu.VMEM((_NBUF, _CHUNK, d_model), jnp.float32),
            pltpu.SemaphoreType.DMA((_NBUF,)),
            pltpu.SemaphoreType.DMA((_NBUF,)),
        ],
    )
    def emb(idx_hbm, table_hbm, out_hbm, idx_v, rows_v, gsem, osem):
        wid = lax.axis_index("s") * 2 + lax.axis_index("c")
        base = wid * per_worker

        def gather_desc(c, b):
            idx_slice = idx_v.at[pl.ds(c * _CHUNK, _CHUNK)]
            return pltpu.make_async_copy(
                table_hbm.at[idx_slice], rows_v.at[b], gsem.at[b])

        def write_desc(c, b):
            dst = out_hbm.at[pl.ds(base + c * _CHUNK, _CHUNK)]
            return pltpu.make_async_copy(rows_v.at[b], dst, osem.at[b])

        pltpu.sync_copy(idx_hbm.at[pl.ds(base, per_worker)], idx_v)

        for c in range(_GLAG):
            gather_desc(c, c % _NBUF).start()

        for w in range(_GLAG):
            gather_desc(w, w % _NBUF).wait()
            write_desc(w, w % _NBUF).start()
            ng = w + _GLAG
            gather_desc(ng, ng % _NBUF).start()

        @pl.loop(_GLAG, n_chunks - _GLAG, step=_NBUF)
        def _(g):
            for b in range(_NBUF):
                w = g + b
                bw = (_GLAG + b) % _NBUF
                gather_desc(w, bw).wait()
                write_desc(w, bw).start()
                ng = w + _GLAG
                bg = (2 * _GLAG + b) % _NBUF
                write_desc(ng - _NBUF, bg).wait()
                gather_desc(ng, bg).start()

        for w in range(n_chunks - _GLAG, n_chunks):
            gather_desc(w, w % _NBUF).wait()
            write_desc(w, w % _NBUF).start()

        for w in range(n_chunks - _NBUF, n_chunks):
            write_desc(w, w % _NBUF).wait()

    out = emb(idx, table)
    return out.reshape(batch, seq, d_model)


# speedup vs baseline: 1.1910x; 1.1624x over previous
import functools

import jax
import jax.numpy as jnp
from jax import lax
from jax.experimental import pallas as pl
from jax.experimental.pallas import tpu as pltpu
from jax.experimental.pallas import tpu_sc as plsc
from jax.experimental.layout import Layout, with_layout_constraint

_NUM_WORKERS = 32
_CHUNK = 400
_NBUF = 4
_GLAG = 2


def kernel(x, table):
    batch, seq = x.shape
    _, d_model = table.shape
    n = batch * seq
    idx = x.reshape(n).astype(jnp.int32)

    per_worker = n // _NUM_WORKERS
    n_chunks = per_worker // _CHUNK
    assert per_worker * _NUM_WORKERS == n and n_chunks * _CHUNK == per_worker
    assert n_chunks % _NBUF == 0 and n_chunks > 2 * _NBUF

    mesh = plsc.VectorSubcoreMesh(core_axis_name="c", subcore_axis_name="s")

    @functools.partial(
        pl.kernel,
        out_type=jax.ShapeDtypeStruct((batch, seq, d_model), jnp.float32),
        mesh=mesh,
        compiler_params=pltpu.CompilerParams(use_tc_tiling_on_sc=False),
        scratch_types=[
            pltpu.VMEM((per_worker,), jnp.int32),
            pltpu.VMEM((_NBUF, _CHUNK, d_model), jnp.float32),
            pltpu.SemaphoreType.DMA((_NBUF,)),
            pltpu.SemaphoreType.DMA((_NBUF,)),
        ],
    )
    def emb(idx_hbm, table_hbm, out3_hbm, idx_v, rows_v, gsem, osem):
        wid = lax.axis_index("s") * 2 + lax.axis_index("c")
        base = wid * per_worker
        rows_per_chunk = _CHUNK // seq

        def gather_desc(c, b):
            idx_slice = idx_v.at[pl.ds(c * _CHUNK, _CHUNK)]
            return pltpu.make_async_copy(
                table_hbm.at[idx_slice], rows_v.at[b], gsem.at[b])

        def write_descs(c, b):
            b0 = (base + c * _CHUNK) // seq
            return [
                pltpu.make_async_copy(
                    rows_v.at[b, pl.ds(r * seq, seq)],
                    out3_hbm.at[b0 + r], osem.at[b])
                for r in range(rows_per_chunk)
            ]

        pltpu.sync_copy(idx_hbm.at[pl.ds(base, per_worker)], idx_v)

        for c in range(_GLAG):
            gather_desc(c, c % _NBUF).start()

        for w in range(_GLAG):
            gather_desc(w, w % _NBUF).wait()
            for d in write_descs(w, w % _NBUF):
                d.start()
            ng = w + _GLAG
            gather_desc(ng, ng % _NBUF).start()

        @pl.loop(_GLAG, n_chunks - _GLAG, step=_NBUF)
        def _(g):
            for b in range(_NBUF):
                w = g + b
                bw = (_GLAG + b) % _NBUF
                gather_desc(w, bw).wait()
                for d in write_descs(w, bw):
                    d.start()
                ng = w + _GLAG
                bg = (2 * _GLAG + b) % _NBUF
                for d in write_descs(ng - _NBUF, bg):
                    d.wait()
                gather_desc(ng, bg).start()

        for w in range(n_chunks - _GLAG, n_chunks):
            gather_desc(w, w % _NBUF).wait()
            for d in write_descs(w, w % _NBUF):
                d.start()

        for w in range(n_chunks - _NBUF, n_chunks):
            for d in write_descs(w, w % _NBUF):
                d.wait()

    out = emb(idx, table)
    return with_layout_constraint(out, Layout(major_to_minor=(0, 1, 2), tiling=((8,), (1024,))))
